# Initial kernel scaffold; baseline (speedup 1.0000x reference)
import jax
import jax.numpy as jnp
from jax.experimental import pallas as pl


def kernel(x_flat, W):
    raise NotImplementedError("write your pallas kernel here")


# speedup vs baseline: 1.0080x; 1.0080x over previous
import functools

import jax
import jax.numpy as jnp
from jax import lax
from jax.experimental import pallas as pl
from jax.experimental.pallas import tpu as pltpu

TOP_K = 8
Z_LOSS_COEF = 0.001


def _router_body(x_ref, wt_ref, idx_ref, val_ref, probs_ref, zsq_ref):
    tm = x_ref.shape[0]
    e_dim = wt_ref.shape[1]
    logits = jnp.dot(x_ref[...], wt_ref[...], preferred_element_type=jnp.float32)
    m = jnp.max(logits, axis=-1, keepdims=True)
    e = jnp.exp(logits - m)
    s = jnp.sum(e, axis=-1, keepdims=True)
    probs = e / s
    probs_ref[...] = probs

    z = m + jnp.log(s)
    part = jnp.reshape(jnp.sum(z * z), (1, 1))
    @pl.when(pl.program_id(0) == 0)
    def _init():
        zsq_ref[...] = part
    @pl.when(pl.program_id(0) != 0)
    def _acc():
        zsq_ref[...] += part

    iota = lax.broadcasted_iota(jnp.int32, (tm, e_dim), 1)
    p = probs
    vals = []
    idxs = []
    for _ in range(TOP_K):
        mv = jnp.max(p, axis=-1, keepdims=True)
        eq = p == mv
        ii = jnp.min(jnp.where(eq, iota, e_dim), axis=-1, keepdims=True)
        vals.append(mv)
        idxs.append(ii)
        p = jnp.where(iota == ii, -1.0, p)
    tv = jnp.concatenate(vals, axis=-1)
    ti = jnp.concatenate(idxs, axis=-1)
    val_ref[...] = tv / (jnp.sum(tv, axis=-1, keepdims=True) + 1e-9)
    idx_ref[...] = ti


@functools.partial(jax.jit, static_argnames=("tm",))
def _router(x_flat, wt, tm=512):
    t, h = x_flat.shape
    e_dim = wt.shape[1]
    grid = (t // tm,)
    idx, val, probs, zsq = pl.pallas_call(
        _router_body,
        grid=grid,
        in_specs=[
            pl.BlockSpec((tm, h), lambda i: (i, 0)),
            pl.BlockSpec((h, e_dim), lambda i: (0, 0)),
        ],
        out_specs=[
            pl.BlockSpec((tm, TOP_K), lambda i: (i, 0)),
            pl.BlockSpec((tm, TOP_K), lambda i: (i, 0)),
            pl.BlockSpec((tm, e_dim), lambda i: (i, 0)),
            pl.BlockSpec((1, 1), lambda i: (0, 0)),
        ],
        out_shape=[
            jax.ShapeDtypeStruct((t, TOP_K), jnp.int32),
            jax.ShapeDtypeStruct((t, TOP_K), jnp.float32),
            jax.ShapeDtypeStruct((t, e_dim), jnp.float32),
            jax.ShapeDtypeStruct((1, 1), jnp.float32),
        ],
    )(x_flat, wt)
    z_loss = Z_LOSS_COEF * zsq[0, 0] / t
    return idx, val, probs, z_loss


def kernel(x_flat, W):
    return _router(x_flat, W.T)
